# baseline (speedup 1.0000x reference)
import functools

import jax
import jax.numpy as jnp
from jax import lax
from jax.experimental import pallas as pl
from jax.experimental.pallas import tpu as pltpu
from jax.experimental.pallas import tpu_sc as plsc

S, B, W = 50, 1024, 21
V, EC, EW, K = 96, 50, 256, 5
N = S * B
P = W + 2
T = P - K + 1
NB = 1024
ECP = 128
ROWS = P * N

_info = plsc.get_sparse_core_info()
_NW = _info.num_cores * _info.num_subcores
ROWS_PER_W = ROWS // _NW
CH = 800
NCH = ROWS_PER_W // CH


@functools.partial(
    pl.kernel,
    mesh=plsc.VectorSubcoreMesh(core_axis_name="c", subcore_axis_name="s"),
    out_type=jax.ShapeDtypeStruct((ROWS, ECP), jnp.float32),
    scratch_types=[
        pltpu.VMEM((CH,), jnp.int32),
        pltpu.VMEM((CH, ECP), jnp.float32),
        pltpu.SemaphoreType.DMA,
    ],
)
def _sc_gather(idx_hbm, tbl_hbm, out_hbm, idx_v, rows_v, sem):
    wid = lax.axis_index("s") * _info.num_cores + lax.axis_index("c")
    base = wid * ROWS_PER_W

    def chunk(i, carry):
        off = base + i * CH
        pltpu.sync_copy(idx_hbm.at[pl.ds(off, CH)], idx_v)
        pltpu.async_copy(tbl_hbm.at[idx_v], rows_v, sem).wait()
        pltpu.sync_copy(rows_v, out_hbm.at[pl.ds(off, CH)])
        return carry

    lax.fori_loop(0, NCH, chunk, 0)


def _tc_body(emb_ref, wk_ref, cb_ref, wp_ref, bp_ref, wg_ref, bg_ref,
             out_ref):
    emb = emb_ref[...].astype(jnp.bfloat16)
    xwin = jnp.concatenate([emb[k:k + T, :, :EC] for k in range(K)],
                           axis=2).reshape(T * NB, K * EC)
    acc = jax.lax.dot_general(
        xwin, wk_ref[...], (((1,), (0,)), ((), ())),
        preferred_element_type=jnp.float32)
    h = jnp.maximum(jnp.max(acc.reshape(T, NB, EW), axis=0) + cb_ref[...], 0.0)
    xp = jnp.maximum(
        jax.lax.dot_general(h, wp_ref[...], (((1,), (0,)), ((), ())),
                            preferred_element_type=jnp.float32) + bp_ref[...],
        0.0)
    xg = jax.nn.sigmoid(
        jax.lax.dot_general(h, wg_ref[...], (((1,), (0,)), ((), ())),
                            preferred_element_type=jnp.float32) + bg_ref[...])
    out_ref[...] = xg * xp + (1.0 - xg) * h


@jax.jit
def _run(idxf, tblp, wk, cb, wpT, bp, wgT, bg):
    emb = _sc_gather(idxf, tblp)
    emb = emb.reshape(P, N, ECP)
    full = lambda shape: pl.BlockSpec(shape, lambda i: (0,) * len(shape))
    return pl.pallas_call(
        _tc_body,
        grid=(N // NB,),
        in_specs=[
            pl.BlockSpec((P, NB, ECP), lambda i: (0, i, 0)),
            full((K * EC, EW)),
            full((1, EW)),
            full((EW, EW)),
            full((1, EW)),
            full((EW, EW)),
            full((1, EW)),
        ],
        out_specs=pl.BlockSpec((NB, EW), lambda i: (i, 0)),
        out_shape=jax.ShapeDtypeStruct((N, EW), jnp.float32),
    )(emb, wk, cb, wpT, bp, wgT, bg)


def kernel(input, table, conv_w, conv_b, w_proj, b_proj, w_gate, b_gate):
    idxf = jnp.pad(input.reshape(N, W), ((0, 0), (1, 1))).T.reshape(ROWS)
    tblp = jnp.pad(table.at[0].set(0.0),
                   ((0, 0), (0, ECP - EC)))
    wk = conv_w.transpose(2, 1, 0).reshape(K * EC, EW).astype(jnp.bfloat16)
    out = _run(idxf.astype(jnp.int32), tblp, wk, conv_b.reshape(1, EW),
               w_proj.T, b_proj.reshape(1, EW), w_gate.T,
               b_gate.reshape(1, EW))
    return out.reshape(S, B, EW)


# speedup vs baseline: 13.0967x; 13.0967x over previous
import functools

import jax
import jax.numpy as jnp
from jax.experimental import pallas as pl

S, B, W = 50, 1024, 21
V, EC, EW, K = 96, 50, 256, 5
N = S * B
P = W + 2
T = W + 2 - K + 1
NB = 1024


def _body(idx_ref, tbl_ref, wk_ref, cb_ref, wpg_ref, bpg_ref, out_ref):
    idx = idx_ref[...][..., None]
    oh = (idx == jax.lax.broadcasted_iota(jnp.int16, (P, NB, V), 2)
          ).astype(jnp.bfloat16).reshape(P * NB, V)
    emb = jax.lax.dot_general(
        oh, tbl_ref[...], (((1,), (0,)), ((), ())),
        preferred_element_type=jnp.float32).astype(jnp.bfloat16
                                                   ).reshape(P, NB, EC)
    xwin = jnp.concatenate([emb[k:k + T] for k in range(K)],
                           axis=2).reshape(T * NB, K * EC)
    acc = jax.lax.dot_general(
        xwin, wk_ref[...], (((1,), (0,)), ((), ())),
        preferred_element_type=jnp.float32)
    h = jnp.maximum(jnp.max(acc.reshape(T, NB, EW), axis=0) + cb_ref[...], 0.0)
    pg = jax.lax.dot_general(
        h.astype(jnp.bfloat16), wpg_ref[...], (((1,), (0,)), ((), ())),
        preferred_element_type=jnp.float32) + bpg_ref[...]
    xp = jnp.maximum(pg[:, :EW], 0.0)
    xg = jax.nn.sigmoid(pg[:, EW:])
    out_ref[...] = xg * xp + (1.0 - xg) * h


@functools.partial(jax.jit, static_argnames=("interpret",))
def _run(idxp, tbl0, wk, cb, wpg, bpg, interpret=False):
    full = lambda shape: pl.BlockSpec(shape, lambda i: (0,) * len(shape))
    return pl.pallas_call(
        _body,
        grid=(N // NB,),
        in_specs=[
            pl.BlockSpec((P, NB), lambda i: (0, i)),
            full((V, EC)),
            full((K * EC, EW)),
            full((1, EW)),
            full((EW, 2 * EW)),
            full((1, 2 * EW)),
        ],
        out_specs=pl.BlockSpec((NB, EW), lambda i: (i, 0)),
        out_shape=jax.ShapeDtypeStruct((N, EW), jnp.float32),
        interpret=interpret,
    )(idxp, tbl0, wk, cb, wpg, bpg)


def kernel(input, table, conv_w, conv_b, w_proj, b_proj, w_gate, b_gate,
           interpret=False):
    idxp = jnp.pad(input.reshape(N, W), ((0, 0), (1, 1))).T.astype(jnp.int16)
    tbl0 = table.at[0].set(0.0).astype(jnp.bfloat16)
    wk = conv_w.transpose(2, 1, 0).reshape(K * EC, EW).astype(jnp.bfloat16)
    wpg = jnp.concatenate([w_proj.T, w_gate.T], axis=1).astype(jnp.bfloat16)
    bpg = jnp.concatenate([b_proj, b_gate]).reshape(1, 2 * EW)
    out = _run(idxp, tbl0, wk, conv_b.reshape(1, EW), wpg, bpg,
               interpret=interpret)
    return out.reshape(S, B, EW)


# speedup vs baseline: 13.3793x; 1.0216x over previous
import functools

import jax
import jax.numpy as jnp
from jax.experimental import pallas as pl

S, B, W = 50, 1024, 21
V, EC, EW, K = 96, 50, 256, 5
N = S * B
P = W + 2
T = W + 2 - K + 1
NB = 1280


def _body(idx_ref, tbl_ref, wk_ref, cb_ref, wp_ref, bp_ref, wg_ref, bg_ref,
          out_ref):
    idx = idx_ref[...][..., None]
    oh = (idx == jax.lax.broadcasted_iota(jnp.int16, (P, NB, V), 2)
          ).astype(jnp.bfloat16).reshape(P * NB, V)
    emb = jax.lax.dot_general(
        oh, tbl_ref[...], (((1,), (0,)), ((), ())),
        preferred_element_type=jnp.float32).astype(jnp.bfloat16
                                                   ).reshape(P, NB, EC)
    xwin = jnp.concatenate([emb[k:k + T] for k in range(K)],
                           axis=2).reshape(T * NB, K * EC)
    acc = jax.lax.dot_general(
        xwin, wk_ref[...], (((1,), (0,)), ((), ())),
        preferred_element_type=jnp.float32)
    h = jnp.maximum(jnp.max(acc.reshape(T, NB, EW), axis=0) + cb_ref[...], 0.0)
    xp = jnp.maximum(
        jax.lax.dot_general(h, wp_ref[...], (((1,), (0,)), ((), ())),
                            preferred_element_type=jnp.float32) + bp_ref[...],
        0.0)
    xg = jax.nn.sigmoid(
        jax.lax.dot_general(h, wg_ref[...], (((1,), (0,)), ((), ())),
                            preferred_element_type=jnp.float32) + bg_ref[...])
    out_ref[...] = xg * xp + (1.0 - xg) * h


@functools.partial(jax.jit, static_argnames=("interpret",))
def _run(idxp, tbl0, wk, cb, wpT, bp, wgT, bg, interpret=False):
    full = lambda shape: pl.BlockSpec(shape, lambda i: (0,) * len(shape))
    return pl.pallas_call(
        _body,
        grid=(N // NB,),
        in_specs=[
            pl.BlockSpec((P, NB), lambda i: (0, i)),
            full((V, EC)),
            full((K * EC, EW)),
            full((1, EW)),
            full((EW, EW)),
            full((1, EW)),
            full((EW, EW)),
            full((1, EW)),
        ],
        out_specs=pl.BlockSpec((NB, EW), lambda i: (i, 0)),
        out_shape=jax.ShapeDtypeStruct((N, EW), jnp.float32),
        interpret=interpret,
    )(idxp, tbl0, wk, cb, wpT, bp, wgT, bg)


def kernel(input, table, conv_w, conv_b, w_proj, b_proj, w_gate, b_gate,
           interpret=False):
    idxp = jnp.pad(input.reshape(N, W), ((0, 0), (1, 1))).T.astype(jnp.int16)
    tbl0 = table.at[0].set(0.0).astype(jnp.bfloat16)
    wk = conv_w.transpose(2, 1, 0).reshape(K * EC, EW).astype(jnp.bfloat16)
    out = _run(idxp, tbl0, wk, conv_b.reshape(1, EW), w_proj.T,
               b_proj.reshape(1, EW), w_gate.T, b_gate.reshape(1, EW),
               interpret=interpret)
    return out.reshape(S, B, EW)
